# baseline (speedup 1.0000x reference)
import functools

import jax
import jax.numpy as jnp
from jax import lax
from jax.experimental import pallas as pl
from jax.experimental.pallas import tpu as pltpu
from jax.experimental.pallas import tpu_sc as plsc

N = 50000
E = 800000
D = 32

NC = 2
NS = 16
NW = NC * NS

EB = 128
SUB = 4
CHUNK = EB * SUB
E_PAD = 802816
EROWS = E_PAD // EB
PER_W = E_PAD // NW
N_ITER = PER_W // CHUNK
N_ITER2 = EROWS // NS // SUB

N_PAD = 50176
STRIPE = N_PAD // NS

_mesh = plsc.VectorSubcoreMesh(core_axis_name="c", subcore_axis_name="s")
_sc_params = pltpu.CompilerParams(use_tc_tiling_on_sc=False)


def _scale_rows(rows, eab, j):
    @pl.loop(0, EB, step=16)
    def _(e):
        ea16 = eab[j, pl.ds(e, 16)]
        for k in range(16):
            s = ea16[k]
            rows[j, e + k, pl.ds(0, 16)] = rows[j, e + k, pl.ds(0, 16)] * s
            rows[j, e + k, pl.ds(16, 16)] = rows[j, e + k, pl.ds(16, 16)] * s


@functools.partial(
    pl.kernel,
    out_type=jax.ShapeDtypeStruct((NC, N_PAD, D), jnp.float32),
    mesh=_mesh,
    compiler_params=_sc_params,
    scratch_types=[
        pltpu.VMEM_SHARED((N_PAD, D), jnp.float32),
        pltpu.VMEM((SUB, EB), jnp.int32),
        pltpu.VMEM((SUB, EB), jnp.int32),
        pltpu.VMEM((SUB, EB), jnp.float32),
        pltpu.VMEM((SUB, EB, D), jnp.float32),
        pltpu.SemaphoreType.DMA,
        pltpu.SemaphoreType.DMA,
    ],
)
def _sc_pass1(x_hbm, edges_hbm, ea_hbm, zero_hbm, out_hbm,
              acc, srcb, dstb, eab, rows, gsem, ssem):
    cid = lax.axis_index("c")
    sid = lax.axis_index("s")
    wid = cid * NS + sid

    pltpu.sync_copy(zero_hbm, acc.at[pl.ds(sid * STRIPE, STRIPE)])
    plsc.subcore_barrier()

    row_base0 = wid * (PER_W // EB)

    @pl.loop(0, N_ITER)
    def _(it):
        row_base = row_base0 + it * SUB
        pltpu.sync_copy(edges_hbm.at[0].at[pl.ds(row_base, SUB)], srcb)
        pltpu.sync_copy(edges_hbm.at[1].at[pl.ds(row_base, SUB)], dstb)
        pltpu.sync_copy(ea_hbm.at[pl.ds(row_base, SUB)], eab)
        cps = [pltpu.async_copy(x_hbm.at[srcb.at[j]], rows.at[j], gsem)
               for j in range(SUB)]
        for cp in cps:
            cp.wait()

        for j in range(SUB):
            _scale_rows(rows, eab, j)

        cps2 = [pltpu.async_copy(rows.at[j], acc.at[dstb.at[j]], ssem,
                                 add=True) for j in range(SUB)]
        for cp in cps2:
            cp.wait()

    plsc.subcore_barrier()
    pltpu.sync_copy(acc.at[pl.ds(sid * STRIPE, STRIPE)],
                    out_hbm.at[cid].at[pl.ds(sid * STRIPE, STRIPE)])


@functools.partial(
    pl.kernel,
    out_type=jax.ShapeDtypeStruct((NC, N_PAD, D), jnp.float32),
    mesh=_mesh,
    compiler_params=_sc_params,
    scratch_types=[
        pltpu.VMEM_SHARED((N_PAD, D), jnp.float32),
        pltpu.VMEM((SUB, EB), jnp.int32),
        pltpu.VMEM((SUB, EB), jnp.int32),
        pltpu.VMEM((SUB, EB), jnp.float32),
        pltpu.VMEM((SUB, EB, D), jnp.float32),
        pltpu.SemaphoreType.DMA,
        pltpu.SemaphoreType.DMA,
    ],
)
def _sc_pass2(z_hbm, edges_hbm, ea_hbm, zero_hbm, out_hbm,
              acc, srcb, dstb, eab, rows, gsem, ssem):
    cid = lax.axis_index("c")
    sid = lax.axis_index("s")

    pltpu.sync_copy(zero_hbm, acc.at[pl.ds(sid * STRIPE, STRIPE)])
    plsc.subcore_barrier()

    row_base0 = sid * (EROWS // NS)

    @pl.loop(0, N_ITER2)
    def _(it):
        row_base = row_base0 + it * SUB
        pltpu.sync_copy(edges_hbm.at[0].at[pl.ds(row_base, SUB)], srcb)
        pltpu.sync_copy(edges_hbm.at[1].at[pl.ds(row_base, SUB)], dstb)
        pltpu.sync_copy(ea_hbm.at[pl.ds(row_base, SUB)], eab)
        cps = [pltpu.async_copy(z_hbm.at[cid].at[srcb.at[j]], rows.at[j],
                                gsem) for j in range(SUB)]
        for cp in cps:
            cp.wait()

        for j in range(SUB):
            _scale_rows(rows, eab, j)

        cps2 = [pltpu.async_copy(rows.at[j], acc.at[dstb.at[j]], ssem,
                                 add=True) for j in range(SUB)]
        for cp in cps2:
            cp.wait()

    plsc.subcore_barrier()
    pltpu.sync_copy(acc.at[pl.ds(sid * STRIPE, STRIPE)],
                    out_hbm.at[cid].at[pl.ds(sid * STRIPE, STRIPE)])


_BLK = 5000
_GRID = N // _BLK


def _tc1_body(x_ref, w_ref, b_ref, xs_ref):
    xs_ref[...] = (jnp.dot(x_ref[...], w_ref[...],
                           preferred_element_type=jnp.float32) + b_ref[...])


def _tc2_body(xs_ref, r_ref, wn1_ref, wg_ref, bg3_ref, ws2_ref, bs2_ref,
              c_ref, zst_ref, zs_ref):
    f32 = jnp.float32
    agg1 = r_ref[0] + r_ref[1]
    x1 = jax.nn.relu(xs_ref[...]
                     + jnp.dot(agg1, wn1_ref[...], preferred_element_type=f32))
    gates = (jnp.dot(x1, wg_ref[...], preferred_element_type=f32)
             + bg3_ref[...])
    i = jax.nn.sigmoid(gates[:, :D])
    g = jnp.tanh(gates[:, D:2 * D])
    o = jax.nn.sigmoid(gates[:, 2 * D:])
    c = i * g
    h = o * jnp.tanh(c)
    c_ref[...] = c
    zst_ref[0] = x1
    zst_ref[1] = h
    z = jnp.concatenate([x1, h], axis=1)
    zs_ref[...] = (jnp.dot(z, ws2_ref[...], preferred_element_type=f32)
                   + bs2_ref[...])


def _tc3_body(zs_ref, r_ref, wn2_ref, wl1_ref, bl1_ref, wl2_ref, bl2_ref,
              z_ref):
    f32 = jnp.float32
    agg2 = jnp.concatenate([r_ref[0], r_ref[1]], axis=1)
    z2 = zs_ref[...] + jnp.dot(agg2, wn2_ref[...], preferred_element_type=f32)
    z3 = jnp.dot(z2, wl1_ref[...], preferred_element_type=f32) + bl1_ref[...]
    z_ref[...] = (jnp.dot(z3, wl2_ref[...], preferred_element_type=f32)
                  + bl2_ref[...])


def kernel(x, edge_index, edge_attr, Ws1, Wn1, bs1, Wi, bi, Wf, bf, Wg, bg,
           Wo, bo, Ws2, Wn2, bs2, Wl1, bl1, Wl2, bl2):
    f32 = jnp.float32
    i32 = jnp.int32

    n_extra = E_PAD - E
    pad_idx = (jnp.arange(n_extra, dtype=i32) * 61) % N
    edges3 = jnp.concatenate(
        [edge_index.astype(i32), jnp.stack([pad_idx, pad_idx])],
        axis=1).reshape(2, EROWS, EB)
    ea_p = jnp.concatenate([edge_attr,
                            jnp.zeros((n_extra,), f32)]).reshape(EROWS, EB)

    zero2d = jnp.zeros((STRIPE, D), f32)

    parts1 = _sc_pass1(x, edges3, ea_p, zero2d)

    nspec = pl.BlockSpec((_BLK, D), lambda i: (i, 0))
    xs = pl.pallas_call(
        _tc1_body,
        grid=(_GRID,),
        in_specs=[nspec, pl.BlockSpec((D, D), lambda i: (0, 0)),
                  pl.BlockSpec((1, D), lambda i: (0, 0))],
        out_specs=nspec,
        out_shape=jax.ShapeDtypeStruct((N, D), f32),
    )(x, Ws1, bs1.reshape(1, D))

    zspec = pl.BlockSpec((2, _BLK, D), lambda i: (0, i, 0))
    c, zst, zs = pl.pallas_call(
        _tc2_body,
        grid=(_GRID,),
        in_specs=[nspec, zspec,
                  pl.BlockSpec((D, D), lambda i: (0, 0)),
                  pl.BlockSpec((D, 3 * D), lambda i: (0, 0)),
                  pl.BlockSpec((1, 3 * D), lambda i: (0, 0)),
                  pl.BlockSpec((2 * D, 2 * D), lambda i: (0, 0)),
                  pl.BlockSpec((1, 2 * D), lambda i: (0, 0))],
        out_specs=[nspec, zspec, pl.BlockSpec((_BLK, 2 * D), lambda i: (i, 0))],
        out_shape=[jax.ShapeDtypeStruct((N, D), f32),
                   jax.ShapeDtypeStruct((2, N, D), f32),
                   jax.ShapeDtypeStruct((N, 2 * D), f32)],
    )(xs, parts1, Wn1,
      jnp.concatenate([Wi[:D], Wg[:D], Wo[:D]], axis=1),
      jnp.concatenate([bi, bg, bo]).reshape(1, 3 * D),
      Ws2, bs2.reshape(1, 2 * D))

    parts2 = _sc_pass2(zst, edges3, ea_p, zero2d)

    z = pl.pallas_call(
        _tc3_body,
        grid=(_GRID,),
        in_specs=[pl.BlockSpec((_BLK, 2 * D), lambda i: (i, 0)), zspec,
                  pl.BlockSpec((2 * D, 2 * D), lambda i: (0, 0)),
                  pl.BlockSpec((2 * D, D), lambda i: (0, 0)),
                  pl.BlockSpec((1, D), lambda i: (0, 0)),
                  pl.BlockSpec((D, 1), lambda i: (0, 0)),
                  pl.BlockSpec((1, 1), lambda i: (0, 0))],
        out_specs=pl.BlockSpec((_BLK, 1), lambda i: (i, 0)),
        out_shape=jax.ShapeDtypeStruct((N, 1), f32),
    )(zs, parts2, Wn2, Wl1, bl1.reshape(1, D), Wl2, bl2.reshape(1, 1))

    h = zst[1]
    return (z, h, c)


# speedup vs baseline: 1.4569x; 1.4569x over previous
import dataclasses
import functools

import jax
import jax.numpy as jnp
from jax import lax
from jax.experimental import pallas as pl
from jax.experimental.pallas import tpu as pltpu
from jax.experimental.pallas import tpu_sc as plsc

N = 50000
E = 800000
D = 32

NC = 2
NS = 16
NW = NC * NS

EB = 128
SUB = 2
CHUNK = EB * SUB
NCH1 = 99
E_PAD = NW * NCH1 * CHUNK
EROWS = E_PAD // EB
EROWS_A = EROWS + 4
NCH2 = EROWS // NS // SUB

N_PAD = 50176
STRIPE = N_PAD // NS

_mesh = plsc.VectorSubcoreMesh(core_axis_name="c", subcore_axis_name="s")
_sc_params = pltpu.CompilerParams(use_tc_tiling_on_sc=False)


_SC_SCRATCH = [
    pltpu.VMEM_SHARED((N_PAD, D), jnp.float32),
    pltpu.VMEM((3 * SUB, EB, D), jnp.float32),
    pltpu.VMEM((3 * SUB, EB), jnp.int32),
    pltpu.VMEM((3 * SUB, EB), jnp.int32),
    pltpu.VMEM((3 * SUB, EB), jnp.float32),
] + [pltpu.SemaphoreType.DMA] * 15


def _sc_sweep(gsrc, srcs_hbm, dst_hbm, ea_hbm, zero_hbm, out_hbm, acc, rows,
              sring, dring, earing, sems, sid, cid, rb0, n_chunks):
    gsem = sems[0:3]
    ssem = sems[3:6]
    issem = sems[6:9]
    idsem = sems[9:12]
    iesem = sems[12:15]

    pltpu.sync_copy(zero_hbm, acc.at[pl.ds(sid * STRIPE, STRIPE)])
    plsc.subcore_barrier()

    def src_cp(c, v):
        return pltpu.make_async_copy(
            srcs_hbm.at[pl.ds(rb0 + c * SUB, SUB)],
            sring.at[pl.ds(v * SUB, SUB)], issem[v])

    def de_cp(c, v):
        return [pltpu.make_async_copy(
            dst_hbm.at[pl.ds(rb0 + c * SUB, SUB)],
            dring.at[pl.ds(v * SUB, SUB)], idsem[v]),
                pltpu.make_async_copy(
            ea_hbm.at[pl.ds(rb0 + c * SUB, SUB)],
            earing.at[pl.ds(v * SUB, SUB)], iesem[v])]

    def gather_cps(v):
        return [pltpu.make_async_copy(
            gsrc.at[sring.at[v * SUB + j]],
            rows.at[v * SUB + j], gsem[v]) for j in range(SUB)]

    def scatter_cps(v):
        return [pltpu.make_async_copy(
            rows.at[v * SUB + j],
            acc.at[dring.at[v * SUB + j]], ssem[v]) for j in range(SUB)]

    src_cp(0, 0).start()
    src_cp(0, 0).wait()
    src_cp(1, 1).start()
    for cp in de_cp(0, 0):
        cp.start()
    for cp in gather_cps(0):
        cp.start()

    @pl.loop(0, n_chunks // 3)
    def _(t):
        for u in range(3):
            c = 3 * t + u
            nu = (u + 1) % 3
            nnu = (u + 2) % 3
            for cp in gather_cps(u):
                cp.wait()

            @pl.when(c >= 2)
            def _():
                for cp in scatter_cps(nu):
                    cp.wait()

            src_cp(c + 2, nnu).start()
            for cp in de_cp(c + 1, nu):
                cp.start()
            src_cp(c + 1, nu).wait()
            for cp in gather_cps(nu):
                cp.start()
            for cp in de_cp(c, u):
                cp.wait()

            for j in range(SUB):
                b = u * SUB + j
                @pl.loop(0, EB, step=16)
                def _(e):
                    ea16 = earing[b, pl.ds(e, 16)]
                    for k in range(16):
                        s = ea16[k]
                        rows[b, e + k, pl.ds(0, 16)] = (
                            rows[b, e + k, pl.ds(0, 16)] * s)
                        rows[b, e + k, pl.ds(16, 16)] = (
                            rows[b, e + k, pl.ds(16, 16)] * s)

            for cp in scatter_cps(u):
                cp.start(add=True)

    for cp in gather_cps(0):
        cp.wait()
    for cp in scatter_cps(1):
        cp.wait()
    for cp in scatter_cps(2):
        cp.wait()
    src_cp(0, 1).wait()
    for cp in de_cp(0, 0):
        cp.wait()

    plsc.subcore_barrier()
    pltpu.sync_copy(acc.at[pl.ds(sid * STRIPE, STRIPE)],
                    out_hbm.at[cid].at[pl.ds(sid * STRIPE, STRIPE)])


@functools.partial(
    pl.kernel,
    out_type=jax.ShapeDtypeStruct((NC, N_PAD, D), jnp.float32),
    mesh=_mesh,
    compiler_params=_sc_params,
    scratch_types=_SC_SCRATCH,
)
def _sc_pass1(x_hbm, srcs_hbm, dst_hbm, ea_hbm, zero_hbm, out_hbm,
              acc, rows, sring, dring, earing, *sems):
    cid = lax.axis_index("c")
    sid = lax.axis_index("s")
    wid = cid * NS + sid
    rb0 = wid * (NCH1 * SUB)
    _sc_sweep(x_hbm, srcs_hbm, dst_hbm, ea_hbm, zero_hbm, out_hbm, acc, rows,
              sring, dring, earing, sems, sid, cid, rb0, NCH1)


@functools.partial(
    pl.kernel,
    out_type=jax.ShapeDtypeStruct((NC, N_PAD, D), jnp.float32),
    mesh=_mesh,
    compiler_params=_sc_params,
    scratch_types=_SC_SCRATCH,
)
def _sc_pass2(z_hbm, srcs_hbm, dst_hbm, ea_hbm, zero_hbm, out_hbm,
              acc, rows, sring, dring, earing, *sems):
    cid = lax.axis_index("c")
    sid = lax.axis_index("s")
    rb0 = sid * (NCH2 * SUB)
    _sc_sweep(z_hbm.at[cid], srcs_hbm, dst_hbm, ea_hbm, zero_hbm, out_hbm,
              acc, rows, sring, dring, earing, sems, sid, cid, rb0, NCH2)


_BLK = 5000
_GRID = N // _BLK


def _tc1_body(x_ref, w_ref, b_ref, xs_ref):
    xs_ref[...] = (jnp.dot(x_ref[...], w_ref[...],
                           preferred_element_type=jnp.float32) + b_ref[...])


def _tc2_body(xs_ref, r_ref, wn1_ref, wg_ref, bg3_ref, ws2_ref, bs2_ref,
              c_ref, zst_ref, zs_ref):
    f32 = jnp.float32
    agg1 = r_ref[0] + r_ref[1]
    x1 = jax.nn.relu(xs_ref[...]
                     + jnp.dot(agg1, wn1_ref[...], preferred_element_type=f32))
    gates = (jnp.dot(x1, wg_ref[...], preferred_element_type=f32)
             + bg3_ref[...])
    i = jax.nn.sigmoid(gates[:, :D])
    g = jnp.tanh(gates[:, D:2 * D])
    o = jax.nn.sigmoid(gates[:, 2 * D:])
    c = i * g
    h = o * jnp.tanh(c)
    c_ref[...] = c
    zst_ref[0] = x1
    zst_ref[1] = h
    z = jnp.concatenate([x1, h], axis=1)
    zs_ref[...] = (jnp.dot(z, ws2_ref[...], preferred_element_type=f32)
                   + bs2_ref[...])


def _tc3_body(zs_ref, r_ref, wn2_ref, wl1_ref, bl1_ref, wl2_ref, bl2_ref,
              z_ref):
    f32 = jnp.float32
    agg2 = jnp.concatenate([r_ref[0], r_ref[1]], axis=1)
    z2 = zs_ref[...] + jnp.dot(agg2, wn2_ref[...], preferred_element_type=f32)
    z3 = jnp.dot(z2, wl1_ref[...], preferred_element_type=f32) + bl1_ref[...]
    z_ref[...] = (jnp.dot(z3, wl2_ref[...], preferred_element_type=f32)
                  + bl2_ref[...])


def kernel(x, edge_index, edge_attr, Ws1, Wn1, bs1, Wi, bi, Wf, bf, Wg, bg,
           Wo, bo, Ws2, Wn2, bs2, Wl1, bl1, Wl2, bl2):
    f32 = jnp.float32
    i32 = jnp.int32

    n_extra = EROWS_A * EB - E
    pad_idx = (jnp.arange(n_extra, dtype=i32) * 61) % N
    srcs = jnp.concatenate([edge_index[0].astype(i32),
                            pad_idx]).reshape(EROWS_A, EB)
    dst_p = jnp.concatenate([edge_index[1].astype(i32),
                             pad_idx]).reshape(EROWS_A, EB)
    ea_p = jnp.concatenate([edge_attr,
                            jnp.zeros((n_extra,), f32)]).reshape(EROWS_A, EB)

    zero2d = jnp.zeros((STRIPE, D), f32)

    parts1 = _sc_pass1(x, srcs, dst_p, ea_p, zero2d)

    nspec = pl.BlockSpec((_BLK, D), lambda i: (i, 0))
    xs = pl.pallas_call(
        _tc1_body,
        grid=(_GRID,),
        in_specs=[nspec, pl.BlockSpec((D, D), lambda i: (0, 0)),
                  pl.BlockSpec((1, D), lambda i: (0, 0))],
        out_specs=nspec,
        out_shape=jax.ShapeDtypeStruct((N, D), f32),
    )(x, Ws1, bs1.reshape(1, D))

    zspec = pl.BlockSpec((2, _BLK, D), lambda i: (0, i, 0))
    c, zst, zs = pl.pallas_call(
        _tc2_body,
        grid=(_GRID,),
        in_specs=[nspec, zspec,
                  pl.BlockSpec((D, D), lambda i: (0, 0)),
                  pl.BlockSpec((D, 3 * D), lambda i: (0, 0)),
                  pl.BlockSpec((1, 3 * D), lambda i: (0, 0)),
                  pl.BlockSpec((2 * D, 2 * D), lambda i: (0, 0)),
                  pl.BlockSpec((1, 2 * D), lambda i: (0, 0))],
        out_specs=[nspec, zspec, pl.BlockSpec((_BLK, 2 * D), lambda i: (i, 0))],
        out_shape=[jax.ShapeDtypeStruct((N, D), f32),
                   jax.ShapeDtypeStruct((2, N, D), f32),
                   jax.ShapeDtypeStruct((N, 2 * D), f32)],
    )(xs, parts1, Wn1,
      jnp.concatenate([Wi[:D], Wg[:D], Wo[:D]], axis=1),
      jnp.concatenate([bi, bg, bo]).reshape(1, 3 * D),
      Ws2, bs2.reshape(1, 2 * D))

    parts2 = _sc_pass2(zst, srcs, dst_p, ea_p, zero2d)

    z = pl.pallas_call(
        _tc3_body,
        grid=(_GRID,),
        in_specs=[pl.BlockSpec((_BLK, 2 * D), lambda i: (i, 0)), zspec,
                  pl.BlockSpec((2 * D, 2 * D), lambda i: (0, 0)),
                  pl.BlockSpec((2 * D, D), lambda i: (0, 0)),
                  pl.BlockSpec((1, D), lambda i: (0, 0)),
                  pl.BlockSpec((D, 1), lambda i: (0, 0)),
                  pl.BlockSpec((1, 1), lambda i: (0, 0))],
        out_specs=pl.BlockSpec((_BLK, 1), lambda i: (i, 0)),
        out_shape=jax.ShapeDtypeStruct((N, 1), f32),
    )(zs, parts2, Wn2, Wl1, bl1.reshape(1, D), Wl2, bl2.reshape(1, 1))

    h = zst[1]
    return (z, h, c)
